# baseline (speedup 1.0000x reference)
import jax
import jax.numpy as jnp
from jax import lax
from jax.experimental import pallas as pl
from jax.experimental.pallas import tpu as pltpu
from jax.experimental.pallas import tpu_sc as plsc

VOCAB = 128
DIM = 32
WINDOW = 1024
TC_BLK = 16


def _sc_gather_packed(idx, tab_flat, n):
    mesh = plsc.VectorSubcoreMesh(core_axis_name="core",
                                  subcore_axis_name="subcore")

    @pl.kernel(out_type=jax.ShapeDtypeStruct((n // 4, 4 * DIM), jnp.float32),
               mesh=mesh,
               compiler_params=pltpu.CompilerParams(
                   use_tc_tiling_on_sc=False, needs_layout_passes=False),
               scratch_types=[pltpu.VMEM((VOCAB * DIM,), jnp.float32)])
    def gather_kernel(table_hbm, i_hbm, o_hbm, tab_v):
        pltpu.sync_copy(table_hbm, tab_v)
        lanes = lax.iota(jnp.int32, 16)

        def body(i_vmem, o_vmem):
            @pl.loop(0, WINDOW, step=32)
            def _(i0):
                r0 = i0 // 4
                for half in range(2):
                    vbase = i_vmem[0, pl.ds(i0 + 16 * half, 16)] * DIM
                    for v in range(16):
                        u = 16 * half + v
                        a0 = vbase[v] + lanes
                        row = r0 + (u % 8)
                        col = DIM * (u // 8)
                        o_vmem[row, pl.ds(col, 16)] = plsc.load_gather(
                            tab_v, [a0])
                        o_vmem[row, pl.ds(col + 16, 16)] = plsc.load_gather(
                            tab_v, [a0 + 16])

        pltpu.emit_pipeline(
            body,
            grid=(n // WINDOW,),
            in_specs=[pl.BlockSpec((1, WINDOW), lambda i: (0, i))],
            out_specs=[pl.BlockSpec((WINDOW // 4, 4 * DIM),
                                    lambda i: (i, 0))],
            core_axis_name=("core", "subcore"),
            dimension_semantics=(pltpu.PARALLEL,),
        )(i_hbm, o_hbm)

    return gather_kernel(tab_flat, idx)


def kernel(x, table):
    nrows, seq = x.shape
    n = nrows * seq
    idx = x.reshape(1, n).astype(jnp.int32)
    tab_flat = table.astype(jnp.float32).reshape(VOCAB * DIM)

    packed = _sc_gather_packed(idx, tab_flat, n)

    p_rows = TC_BLK * seq // 4

    def fmt(p_ref, o_ref):
        for t in range(p_rows // 8):
            for a in range(4):
                f0 = 32 * t + 8 * a
                o_ref[f0 // seq, pl.ds(f0 % seq, 8), :] = (
                    p_ref[pl.ds(8 * t, 8), pl.ds(DIM * a, DIM)])

    out = pl.pallas_call(
        fmt,
        grid=(nrows // TC_BLK,),
        in_specs=[pl.BlockSpec((p_rows, 4 * DIM), lambda i: (i, 0))],
        out_specs=pl.BlockSpec((TC_BLK, seq, DIM), lambda i: (i, 0, 0)),
        out_shape=jax.ShapeDtypeStruct((nrows, seq, DIM), jnp.float32),
    )(packed)
    return out
